# Initial kernel scaffold; baseline (speedup 1.0000x reference)
import jax
import jax.numpy as jnp
from jax.experimental import pallas as pl


def kernel(x, edge_index, W1, b1, W2, b2):
    raise NotImplementedError("write your pallas kernel here")


# speedup vs baseline: 21.3675x; 21.3675x over previous
import functools

import jax
import jax.numpy as jnp
from jax import lax
from jax.experimental import pallas as pl
from jax.experimental.pallas import tpu as pltpu
from jax.experimental.pallas import tpu_sc as plsc

N_NODES = 10000
IN_CH = 128
HID_CH = 64
N_EDGES = 320000

NC = 2
NS = 16
NW = NC * NS
LANES = 16

EPT = N_EDGES // NW
CHUNK = 80
NCHUNK = EPT // CHUNK
N_PAD = 10240
RPT = N_PAD // NS


def _sc_mesh():
    return plsc.VectorSubcoreMesh(
        core_axis_name="c", subcore_axis_name="s", num_cores=NC,
        num_subcores=NS)


def _hist_body(dst_hbm, hist_hbm, dst_v, hist_v):
    cid = lax.axis_index("c")
    sid = lax.axis_index("s")
    wid = cid * NS + sid
    pltpu.sync_copy(dst_hbm.at[pl.ds(wid * EPT, EPT)], dst_v)

    zeros = jnp.zeros((LANES,), jnp.float32)

    def zero_step(i, _):
        hist_v[pl.ds(i * LANES, LANES)] = zeros
        return 0

    lax.fori_loop(0, N_NODES // LANES, zero_step, 0)

    ones = jnp.ones((LANES,), jnp.float32)

    def add_step(i, _):
        idx = dst_v[pl.ds(i * LANES, LANES)]
        plsc.addupdate_scatter(hist_v, [idx], ones)
        return 0

    lax.fori_loop(0, EPT // LANES, add_step, 0)
    pltpu.sync_copy(hist_v, hist_hbm.at[pl.ds(wid * N_NODES, N_NODES)])


def _degree_hists(dst32):
    k = pl.kernel(
        _hist_body,
        out_type=jax.ShapeDtypeStruct((NW * N_NODES,), jnp.float32),
        mesh=_sc_mesh(),
        scratch_types=[
            pltpu.VMEM((EPT,), jnp.int32),
            pltpu.VMEM((N_NODES,), jnp.float32),
        ],
        compiler_params=pltpu.CompilerParams(needs_layout_passes=False),
    )
    return k(dst32).reshape(NW, N_NODES)


def _scatter_body(hs_hbm, src_hbm, dst_hbm, zero_hbm, part_hbm,
                  acc, src_v, dst_v, rows0, rows1, gsem, ssem):
    cid = lax.axis_index("c")
    sid = lax.axis_index("s")
    wid = cid * NS + sid
    pltpu.sync_copy(src_hbm.at[wid], src_v)
    pltpu.sync_copy(dst_hbm.at[wid], dst_v)
    pltpu.sync_copy(zero_hbm, acc.at[pl.ds(sid * RPT, RPT)])
    plsc.subcore_barrier()

    del rows1, gsem, ssem

    def step(j, _):
        pltpu.sync_copy(hs_hbm.at[src_v.at[j]], rows0)
        pltpu.sync_copy(rows0, acc.at[dst_v.at[j]], add=True)
        return 0

    lax.fori_loop(0, NCHUNK, step, 0)
    plsc.subcore_barrier()
    pltpu.sync_copy(acc.at[pl.ds(sid * RPT, RPT)],
                    part_hbm.at[cid, pl.ds(sid * RPT, RPT)])


def _edge_scatter(hs, src3, dst3, width):
    k = pl.kernel(
        _scatter_body,
        out_type=jax.ShapeDtypeStruct((NC, N_PAD, width), jnp.float32),
        mesh=_sc_mesh(),
        scratch_types=[
            pltpu.VMEM_SHARED((N_PAD, width), jnp.float32),
            pltpu.VMEM((NCHUNK, CHUNK), jnp.int32),
            pltpu.VMEM((NCHUNK, CHUNK), jnp.int32),
            pltpu.VMEM((CHUNK, width), jnp.float32),
            pltpu.VMEM((CHUNK, width), jnp.float32),
            pltpu.SemaphoreType.DMA,
            pltpu.SemaphoreType.DMA,
        ],
        compiler_params=pltpu.CompilerParams(
            needs_layout_passes=False, use_tc_tiling_on_sc=False),
    )
    zero = jnp.zeros((RPT, width), jnp.float32)
    return k(hs, src3, dst3, zero)[:, :N_NODES]


ROWS_BLK = 1000
N_BLKS = N_NODES // ROWS_BLK


def _k1_body(x_ref, hist_ref, w_ref, hs_ref, dis_ref):
    deg = jnp.sum(hist_ref[...], axis=1) + 1.0
    dis = lax.rsqrt(deg)
    dis_ref[...] = dis[:, None]
    h = jnp.dot(x_ref[...], w_ref[...], preferred_element_type=jnp.float32)
    hs_ref[...] = h * dis[:, None]


def _k1(x, hists_t, W1):
    return pl.pallas_call(
        _k1_body,
        grid=(N_BLKS,),
        in_specs=[
            pl.BlockSpec((ROWS_BLK, IN_CH), lambda i: (i, 0)),
            pl.BlockSpec((ROWS_BLK, NW), lambda i: (i, 0)),
            pl.BlockSpec((IN_CH, HID_CH), lambda i: (0, 0)),
        ],
        out_specs=[
            pl.BlockSpec((ROWS_BLK, HID_CH), lambda i: (i, 0)),
            pl.BlockSpec((ROWS_BLK, 1), lambda i: (i, 0)),
        ],
        out_shape=[
            jax.ShapeDtypeStruct((N_NODES, HID_CH), jnp.float32),
            jax.ShapeDtypeStruct((N_NODES, 1), jnp.float32),
        ],
    )(x, hists_t, W1)


def _k3_body(p_ref, hs_ref, dis_ref, b_ref, w_ref, out_ref):
    dis = dis_ref[...]
    s = p_ref[0] + p_ref[1] + hs_ref[...]
    t = jnp.maximum(s * dis + b_ref[...], 0.0)
    h = jnp.dot(t, w_ref[...], preferred_element_type=jnp.float32)
    out_ref[...] = h * dis


def _k3(part, hs1, dis, b1, W2):
    return pl.pallas_call(
        _k3_body,
        grid=(N_BLKS,),
        in_specs=[
            pl.BlockSpec((NC, ROWS_BLK, HID_CH), lambda i: (0, i, 0)),
            pl.BlockSpec((ROWS_BLK, HID_CH), lambda i: (i, 0)),
            pl.BlockSpec((ROWS_BLK, 1), lambda i: (i, 0)),
            pl.BlockSpec((1, HID_CH), lambda i: (0, 0)),
            pl.BlockSpec((HID_CH, IN_CH), lambda i: (0, 0)),
        ],
        out_specs=pl.BlockSpec((ROWS_BLK, IN_CH), lambda i: (i, 0)),
        out_shape=jax.ShapeDtypeStruct((N_NODES, IN_CH), jnp.float32),
    )(part, hs1, dis, b1.reshape(1, HID_CH), W2)


def _k5_body(p_ref, hs_ref, dis_ref, b_ref, out_ref):
    s = p_ref[0] + p_ref[1] + hs_ref[...]
    out_ref[...] = jnp.tanh(s * dis_ref[...] + b_ref[...])


def _k5(part, hs2, dis, b2):
    return pl.pallas_call(
        _k5_body,
        grid=(N_BLKS,),
        in_specs=[
            pl.BlockSpec((NC, ROWS_BLK, IN_CH), lambda i: (0, i, 0)),
            pl.BlockSpec((ROWS_BLK, IN_CH), lambda i: (i, 0)),
            pl.BlockSpec((ROWS_BLK, 1), lambda i: (i, 0)),
            pl.BlockSpec((1, IN_CH), lambda i: (0, 0)),
        ],
        out_specs=pl.BlockSpec((ROWS_BLK, IN_CH), lambda i: (i, 0)),
        out_shape=jax.ShapeDtypeStruct((N_NODES, IN_CH), jnp.float32),
    )(part, hs2, dis, b2.reshape(1, IN_CH))


@jax.jit
def kernel(x, edge_index, W1, b1, W2, b2):
    src = edge_index[0].astype(jnp.int32)
    dst = edge_index[1].astype(jnp.int32)
    src3 = src.reshape(NW, NCHUNK, CHUNK)
    dst3 = dst.reshape(NW, NCHUNK, CHUNK)

    hists_t = _degree_hists(dst).T

    hs1, dis = _k1(x, hists_t, W1)
    part1 = _edge_scatter(hs1, src3, dst3, HID_CH)
    hs2 = _k3(part1, hs1, dis, b1, W2)
    part2 = _edge_scatter(hs2, src3, dst3, IN_CH)
    return _k5(part2, hs2, dis, b2)
